# Initial kernel scaffold; baseline (speedup 1.0000x reference)
import jax
import jax.numpy as jnp
from jax.experimental import pallas as pl


def kernel(hm, wh, reg, target_sizes):
    raise NotImplementedError("write your pallas kernel here")


# speedup vs baseline: 4.5998x; 4.5998x over previous
import functools

import jax
import jax.numpy as jnp
from jax.experimental import pallas as pl
from jax.experimental.pallas import tpu as pltpu

IOU_THR = 0.4
OUT_THRESH = 0.1
KPAD = 1024


def _heat_kernel(hm_ref, out_ref):
    x = hm_ref[...]
    s = 1.0 / (1.0 + jnp.exp(-x))
    s = jnp.clip(s, 1e-4, 1.0 - 1e-4)
    z = jnp.zeros_like(s[:, :, :1])
    xl = jnp.concatenate([s[:, :, 1:], z], axis=2)
    xr = jnp.concatenate([z, s[:, :, :-1]], axis=2)
    m = jnp.maximum(jnp.maximum(xl, xr), s)
    zr = jnp.zeros_like(m[:, :1, :])
    mu = jnp.concatenate([m[:, 1:, :], zr], axis=1)
    md = jnp.concatenate([zr, m[:, :-1, :]], axis=1)
    hmax = jnp.maximum(jnp.maximum(mu, md), m)
    out_ref[...] = s * (hmax == s).astype(jnp.float32)


def _nms_kernel(sc_ref, inds_ref, rx_ref, ry_ref, bw_ref, bh_ref, cx_ref,
                cy_ref, scale_ref, x1o, y1o, x2o, y2o, sco,
                x1s, y1s, x2s, y2s, ars, kps, W, H, K):
    scores = sc_ref[...]
    inds = inds_ref[...]
    ys = (inds // W).astype(jnp.float32)
    xs = (inds % W).astype(jnp.float32)
    xs = xs + rx_ref[...]
    ys = ys + ry_ref[...]
    bw = bw_ref[...]
    bh = bh_ref[...]
    x1 = xs - bw * 0.5
    y1 = ys - bh * 0.5
    x2 = xs + bw * 0.5
    y2 = ys + bh * 0.5
    cx = cx_ref[...]
    cy = cy_ref[...]
    scale = scale_ref[...]
    x1 = (x1 - W / 2.0) * scale + cx
    x2 = (x2 - W / 2.0) * scale + cx
    y1 = (y1 - H / 2.0) * scale + cy
    y2 = (y2 - H / 2.0) * scale + cy
    x1s[...] = x1
    y1s[...] = y1
    x2s[...] = x2
    y2s[...] = y2
    ars[...] = jnp.clip(x2 - x1, 0.0) * jnp.clip(y2 - y1, 0.0)
    kps[...] = (scores >= OUT_THRESH).astype(jnp.float32)
    lane = jax.lax.broadcasted_iota(jnp.int32, scores.shape, 1)
    lane128 = jax.lax.broadcasted_iota(jnp.int32, (scores.shape[0], 128), 1)

    def body(i, carry):
        cbase = pl.multiple_of((i // 128) * 128, 128)
        li = i % 128
        sel = lane128 == li

        def ext(ref):
            c = ref[:, pl.ds(cbase, 128)]
            return jnp.max(jnp.where(sel, c, -3.4e38), axis=1, keepdims=True)

        kx1 = ext(x1s)
        ky1 = ext(y1s)
        kx2 = ext(x2s)
        ky2 = ext(y2s)
        kar = ext(ars)
        kpi = ext(kps)
        ix1 = jnp.maximum(x1s[...], kx1)
        iy1 = jnp.maximum(y1s[...], ky1)
        ix2 = jnp.minimum(x2s[...], kx2)
        iy2 = jnp.minimum(y2s[...], ky2)
        inter = jnp.clip(ix2 - ix1, 0.0) * jnp.clip(iy2 - iy1, 0.0)
        union = jnp.maximum(ars[...] + kar - inter, 1e-6)
        sup = (kpi > 0.0) & (inter > IOU_THR * union) & (lane > i)
        kps[...] = kps[...] * (1.0 - sup.astype(jnp.float32))
        return carry

    jax.lax.fori_loop(0, K, body, 0)
    kp = kps[...]
    x1o[...] = x1 * kp
    y1o[...] = y1 * kp
    x2o[...] = x2 * kp
    y2o[...] = y2 * kp
    sco[...] = scores * kp


def kernel(hm, wh, reg, target_sizes):
    B, C, H, W = hm.shape
    K = 1000
    HW = H * W
    heat = pl.pallas_call(
        _heat_kernel,
        out_shape=jax.ShapeDtypeStruct((B, H, W), jnp.float32),
    )(hm.reshape(B, H, W))

    scores, inds = jax.lax.top_k(heat.reshape(B, HW), K)
    pad = KPAD - K
    scores = jnp.pad(scores, ((0, 0), (0, pad)))
    inds = jnp.pad(inds, ((0, 0), (0, pad)))

    regf = reg.reshape(B, 2, HW)
    whf = wh.reshape(B, 2, HW)
    rx = jnp.take_along_axis(regf[:, 0], inds, axis=1)
    ry = jnp.take_along_axis(regf[:, 1], inds, axis=1)
    bw = jnp.take_along_axis(whf[:, 0], inds, axis=1)
    bh = jnp.take_along_axis(whf[:, 1], inds, axis=1)

    ts = target_sizes.astype(jnp.float32)
    cx = ts[:, 1:2] / 2.0
    cy = ts[:, 0:1] / 2.0
    scale = jnp.maximum(ts[:, 0:1], ts[:, 1:2]) / float(W)

    out_sh = jax.ShapeDtypeStruct((B, KPAD), jnp.float32)
    scr = pltpu.VMEM((B, KPAD), jnp.float32)
    x1, y1, x2, y2, sc = pl.pallas_call(
        functools.partial(_nms_kernel, W=W, H=H, K=K),
        out_shape=[out_sh] * 5,
        scratch_shapes=[scr] * 6,
    )(scores, inds, rx, ry, bw, bh, cx, cy, scale)

    out = jnp.stack([x1[:, :K], y1[:, :K], x2[:, :K], y2[:, :K], sc[:, :K]],
                    axis=-1)
    return out


# speedup vs baseline: 7.8062x; 1.6971x over previous
import functools

import jax
import jax.numpy as jnp
from jax.experimental import pallas as pl
from jax.experimental.pallas import tpu as pltpu

IOU_THR = 0.4
OUT_THRESH = 0.1
KPAD = 1024
ONE_BITS = 0x3F800000


def _heat_kernel(hm_ref, out_ref, thr_ref, K):
    x = hm_ref[...]
    s = 1.0 / (1.0 + jnp.exp(-x))
    s = jnp.clip(s, 1e-4, 1.0 - 1e-4)
    z = jnp.zeros_like(s[:, :, :1])
    xl = jnp.concatenate([s[:, :, 1:], z], axis=2)
    xr = jnp.concatenate([z, s[:, :, :-1]], axis=2)
    m = jnp.maximum(jnp.maximum(xl, xr), s)
    zr = jnp.zeros_like(m[:, :1, :])
    mu = jnp.concatenate([m[:, 1:, :], zr], axis=1)
    md = jnp.concatenate([zr, m[:, :-1, :]], axis=1)
    hmax = jnp.maximum(jnp.maximum(mu, md), m)
    sup = s * (hmax == s).astype(jnp.float32)
    out_ref[...] = sup

    B = sup.shape[0]
    bits = jax.lax.bitcast_convert_type(sup, jnp.int32)

    def bs_body(_, lohi):
        lo, hi = lohi
        mid = (lo + hi) // 2
        cnt = jnp.sum((bits >= mid).astype(jnp.int32), axis=(1, 2),
                      keepdims=True)
        ok = cnt >= K
        return jnp.where(ok, mid, lo), jnp.where(ok, hi, mid)

    lo0 = jnp.zeros((B, 1, 1), jnp.int32)
    hi0 = jnp.full((B, 1, 1), ONE_BITS, jnp.int32)
    lo, _ = jax.lax.fori_loop(0, 30, bs_body, (lo0, hi0))
    vk = jax.lax.bitcast_convert_type(lo, jnp.float32)
    thr_ref[...] = jnp.maximum(vk, OUT_THRESH).reshape(B, 1)


def _compact_kernel(sv_ref, rx_ref, ry_ref, bw_ref, bh_ref, thr_ref,
                    ds_o, drx_o, dry_o, dbw_o, dbh_o, dix_o, acc_ref):
    B, R, L = sv_ref.shape
    sv = sv_ref[...]
    thr = thr_ref[...].reshape(B, 1, 1)
    mask = sv >= thr
    mi = mask.astype(jnp.int32)
    M = mi
    sh = 1
    while sh < R:
        zpad = jnp.zeros((B, sh, L), jnp.int32)
        M = M + jnp.concatenate([zpad, M[:, :-sh, :]], axis=1)
        sh *= 2
    cnt = M[:, R - 1, :]
    jmax = jnp.max(cnt)
    wl = mi
    sh = 1
    while sh < L:
        zpad2 = jnp.zeros((B, R, sh), jnp.int32)
        wl = wl + jnp.concatenate([zpad2, wl[:, :, :-sh]], axis=2)
        sh *= 2
    wl = wl - mi
    rowcnt = jnp.sum(mi, axis=2, keepdims=True)
    rb = rowcnt
    sh = 1
    while sh < R:
        zpad3 = jnp.zeros((B, sh, 1), jnp.int32)
        rb = rb + jnp.concatenate([zpad3, rb[:, :-sh, :]], axis=1)
        sh *= 2
    rb = rb - rowcnt
    dmap = (rb + wl).astype(jnp.float32)

    r3 = jax.lax.broadcasted_iota(jnp.int32, (B, R, L), 1)
    l3 = jax.lax.broadcasted_iota(jnp.int32, (B, R, L), 2)
    lin = (r3 * L + l3).astype(jnp.float32)
    eye = (jax.lax.broadcasted_iota(jnp.int32, (B, B), 0) ==
           jax.lax.broadcasted_iota(jnp.int32, (B, B), 1)).astype(jnp.float32)
    kio = jax.lax.broadcasted_iota(jnp.int32, (1, KPAD), 1).astype(jnp.float32)
    dn = (((0,), (0,)), ((), ()))

    acc_ref[...] = jnp.zeros_like(acc_ref)

    def body(j, carry):
        sel = mask & (M == j + 1)

        def rv(p):
            return jnp.sum(jnp.where(sel, p, 0.0), axis=1)

        rs, rrx, rry, rbw, rbh, rlin = (rv(sv), rv(rx_ref[...]),
                                        rv(ry_ref[...]), rv(bw_ref[...]),
                                        rv(bh_ref[...]), rv(lin))
        d = rv(dmap)
        dT = jax.lax.dot_general(d, eye, dn,
                                 preferred_element_type=jnp.float32,
                                 precision=jax.lax.Precision.HIGHEST)
        for b in range(B):
            onehot = (jnp.abs(dT[:, b:b + 1] - kio) < 0.5).astype(jnp.float32)
            P = jnp.concatenate([rs[b:b + 1], rrx[b:b + 1], rry[b:b + 1],
                                 rbw[b:b + 1], rbh[b:b + 1], rlin[b:b + 1]],
                                axis=0)
            res = jnp.dot(P, onehot, preferred_element_type=jnp.float32,
                          precision=jax.lax.Precision.HIGHEST)
            acc_ref[b, 0:6, :] = acc_ref[b, 0:6, :] + res
        return carry

    jax.lax.fori_loop(0, jmax, body, 0)
    ds_o[...] = acc_ref[:, 0, :]
    drx_o[...] = acc_ref[:, 1, :]
    dry_o[...] = acc_ref[:, 2, :]
    dbw_o[...] = acc_ref[:, 3, :]
    dbh_o[...] = acc_ref[:, 4, :]
    dix_o[...] = acc_ref[:, 5, :]


def _nms_kernel(sc_ref, inds_ref, rx_ref, ry_ref, bw_ref, bh_ref, cx_ref,
                cy_ref, scale_ref, x1o, y1o, x2o, y2o, sco,
                x1s, y1s, x2s, y2s, ars, kps, W, H, K):
    scores = sc_ref[...]
    inds = inds_ref[...]
    ys = (inds // W).astype(jnp.float32)
    xs = (inds % W).astype(jnp.float32)
    xs = xs + rx_ref[...]
    ys = ys + ry_ref[...]
    bw = bw_ref[...]
    bh = bh_ref[...]
    x1 = xs - bw * 0.5
    y1 = ys - bh * 0.5
    x2 = xs + bw * 0.5
    y2 = ys + bh * 0.5
    cx = cx_ref[...]
    cy = cy_ref[...]
    scale = scale_ref[...]
    x1 = (x1 - W / 2.0) * scale + cx
    x2 = (x2 - W / 2.0) * scale + cx
    y1 = (y1 - H / 2.0) * scale + cy
    y2 = (y2 - H / 2.0) * scale + cy
    x1s[...] = x1
    y1s[...] = y1
    x2s[...] = x2
    y2s[...] = y2
    ars[...] = jnp.clip(x2 - x1, 0.0) * jnp.clip(y2 - y1, 0.0)
    kps[...] = (scores >= OUT_THRESH).astype(jnp.float32)
    lane = jax.lax.broadcasted_iota(jnp.int32, scores.shape, 1)
    lane128 = jax.lax.broadcasted_iota(jnp.int32, (scores.shape[0], 128), 1)

    def body(i, carry):
        cbase = pl.multiple_of((i // 128) * 128, 128)
        li = i % 128
        sel = lane128 == li

        def ext(ref):
            c = ref[:, pl.ds(cbase, 128)]
            return jnp.max(jnp.where(sel, c, -3.4e38), axis=1, keepdims=True)

        kx1 = ext(x1s)
        ky1 = ext(y1s)
        kx2 = ext(x2s)
        ky2 = ext(y2s)
        kar = ext(ars)
        kpi = ext(kps)
        ix1 = jnp.maximum(x1s[...], kx1)
        iy1 = jnp.maximum(y1s[...], ky1)
        ix2 = jnp.minimum(x2s[...], kx2)
        iy2 = jnp.minimum(y2s[...], ky2)
        inter = jnp.clip(ix2 - ix1, 0.0) * jnp.clip(iy2 - iy1, 0.0)
        union = jnp.maximum(ars[...] + kar - inter, 1e-6)
        sup = (kpi > 0.0) & (inter > IOU_THR * union) & (lane > i)
        kps[...] = kps[...] * (1.0 - sup.astype(jnp.float32))
        return carry

    jax.lax.fori_loop(0, K, body, 0)
    kp = kps[...]
    x1o[...] = x1 * kp
    y1o[...] = y1 * kp
    x2o[...] = x2 * kp
    y2o[...] = y2 * kp
    sco[...] = scores * kp


def kernel(hm, wh, reg, target_sizes):
    B, C, H, W = hm.shape
    K = 1000
    HW = H * W
    RPAD = 328

    heat, thr = pl.pallas_call(
        functools.partial(_heat_kernel, K=K),
        out_shape=[jax.ShapeDtypeStruct((B, H, W), jnp.float32),
                   jax.ShapeDtypeStruct((B, 1), jnp.float32)],
    )(hm.reshape(B, H, W))

    def to_grid(a):
        flat = a.reshape(B, HW)
        flat = jnp.pad(flat, ((0, 0), (0, RPAD * 128 - HW)))
        return flat.reshape(B, RPAD, 128)

    regf = reg.reshape(B, 2, HW)
    whf = wh.reshape(B, 2, HW)
    ds, drx, dry, dbw, dbh, dix = pl.pallas_call(
        _compact_kernel,
        out_shape=[jax.ShapeDtypeStruct((B, KPAD), jnp.float32)] * 6,
        scratch_shapes=[pltpu.VMEM((B, 8, KPAD), jnp.float32)],
    )(to_grid(heat), to_grid(regf[:, 0]), to_grid(regf[:, 1]),
      to_grid(whf[:, 0]), to_grid(whf[:, 1]), thr)

    scores, pos = jax.lax.top_k(ds, K)
    pad = KPAD - K

    def takep(a):
        return jnp.pad(jnp.take_along_axis(a, pos, axis=1),
                       ((0, 0), (0, pad)))

    scores = jnp.pad(scores, ((0, 0), (0, pad)))
    inds = takep(dix).astype(jnp.int32)
    rx, ry, bw, bh = takep(drx), takep(dry), takep(dbw), takep(dbh)

    ts = target_sizes.astype(jnp.float32)
    cx = ts[:, 1:2] / 2.0
    cy = ts[:, 0:1] / 2.0
    scale = jnp.maximum(ts[:, 0:1], ts[:, 1:2]) / float(W)

    out_sh = jax.ShapeDtypeStruct((B, KPAD), jnp.float32)
    scr = pltpu.VMEM((B, KPAD), jnp.float32)
    x1, y1, x2, y2, sc = pl.pallas_call(
        functools.partial(_nms_kernel, W=W, H=H, K=K),
        out_shape=[out_sh] * 5,
        scratch_shapes=[scr] * 6,
    )(scores, inds, rx, ry, bw, bh, cx, cy, scale)

    out = jnp.stack([x1[:, :K], y1[:, :K], x2[:, :K], y2[:, :K], sc[:, :K]],
                    axis=-1)
    return out


# speedup vs baseline: 8.0406x; 1.0300x over previous
import functools

import jax
import jax.numpy as jnp
from jax.experimental import pallas as pl
from jax.experimental.pallas import tpu as pltpu

IOU_THR = 0.4
OUT_THRESH = 0.1
KPAD = 1024
ONE_BITS = 0x3F800000


def _heat_kernel(hm_ref, out_ref, thr_ref, K):
    x = hm_ref[...]
    s = 1.0 / (1.0 + jnp.exp(-x))
    s = jnp.clip(s, 1e-4, 1.0 - 1e-4)
    z = jnp.zeros_like(s[:, :, :1])
    xl = jnp.concatenate([s[:, :, 1:], z], axis=2)
    xr = jnp.concatenate([z, s[:, :, :-1]], axis=2)
    m = jnp.maximum(jnp.maximum(xl, xr), s)
    zr = jnp.zeros_like(m[:, :1, :])
    mu = jnp.concatenate([m[:, 1:, :], zr], axis=1)
    md = jnp.concatenate([zr, m[:, :-1, :]], axis=1)
    hmax = jnp.maximum(jnp.maximum(mu, md), m)
    sup = s * (hmax == s).astype(jnp.float32)
    out_ref[...] = sup

    B = sup.shape[0]
    bits = jax.lax.bitcast_convert_type(sup, jnp.int32)

    def bs_body(_, lohi):
        lo, hi = lohi
        mid = (lo + hi) // 2
        cnt = jnp.sum((bits >= mid).astype(jnp.int32), axis=(1, 2),
                      keepdims=True)
        ok = cnt >= K
        return jnp.where(ok, mid, lo), jnp.where(ok, hi, mid)

    lo0 = jnp.zeros((B, 1, 1), jnp.int32)
    hi0 = jnp.full((B, 1, 1), ONE_BITS, jnp.int32)
    lo, _ = jax.lax.fori_loop(0, 30, bs_body, (lo0, hi0))
    vk = jax.lax.bitcast_convert_type(lo, jnp.float32)
    thr_ref[...] = jnp.maximum(vk, OUT_THRESH).reshape(B, 1)


def _compact_kernel(sv_ref, rx_ref, ry_ref, bw_ref, bh_ref, thr_ref,
                    ds_o, drx_o, dry_o, dbw_o, dbh_o, dix_o, acc_ref):
    B, R, L = sv_ref.shape
    sv = sv_ref[...]
    thr = thr_ref[...].reshape(B, 1, 1)
    mask = sv >= thr
    mi = mask.astype(jnp.int32)
    M = mi
    sh = 1
    while sh < R:
        zpad = jnp.zeros((B, sh, L), jnp.int32)
        M = M + jnp.concatenate([zpad, M[:, :-sh, :]], axis=1)
        sh *= 2
    cnt = M[:, R - 1, :]
    jmax = jnp.max(cnt)
    wl = mi
    sh = 1
    while sh < L:
        zpad2 = jnp.zeros((B, R, sh), jnp.int32)
        wl = wl + jnp.concatenate([zpad2, wl[:, :, :-sh]], axis=2)
        sh *= 2
    wl = wl - mi
    rowcnt = jnp.sum(mi, axis=2, keepdims=True)
    rb = rowcnt
    sh = 1
    while sh < R:
        zpad3 = jnp.zeros((B, sh, 1), jnp.int32)
        rb = rb + jnp.concatenate([zpad3, rb[:, :-sh, :]], axis=1)
        sh *= 2
    rb = rb - rowcnt
    dmap = (rb + wl).astype(jnp.float32)

    r3 = jax.lax.broadcasted_iota(jnp.int32, (B, R, L), 1)
    l3 = jax.lax.broadcasted_iota(jnp.int32, (B, R, L), 2)
    lin = (r3 * L + l3).astype(jnp.float32)
    eye = (jax.lax.broadcasted_iota(jnp.int32, (B, B), 0) ==
           jax.lax.broadcasted_iota(jnp.int32, (B, B), 1)).astype(jnp.float32)
    kio = jax.lax.broadcasted_iota(jnp.int32, (1, KPAD), 1).astype(jnp.float32)
    dn = (((0,), (0,)), ((), ()))

    acc_ref[...] = jnp.zeros_like(acc_ref)

    def body(j, carry):
        sel = mask & (M == j + 1)

        def rv(p):
            return jnp.sum(jnp.where(sel, p, 0.0), axis=1)

        rs, rrx, rry, rbw, rbh, rlin = (rv(sv), rv(rx_ref[...]),
                                        rv(ry_ref[...]), rv(bw_ref[...]),
                                        rv(bh_ref[...]), rv(lin))
        d = rv(dmap)
        dT = jax.lax.dot_general(d, eye, dn,
                                 preferred_element_type=jnp.float32,
                                 precision=jax.lax.Precision.HIGHEST)
        for b in range(B):
            onehot = (jnp.abs(dT[:, b:b + 1] - kio) < 0.5).astype(jnp.float32)
            P = jnp.concatenate([rs[b:b + 1], rrx[b:b + 1], rry[b:b + 1],
                                 rbw[b:b + 1], rbh[b:b + 1], rlin[b:b + 1]],
                                axis=0)
            res = jnp.dot(P, onehot, preferred_element_type=jnp.float32,
                          precision=jax.lax.Precision.HIGHEST)
            acc_ref[b, 0:6, :] = acc_ref[b, 0:6, :] + res
        return carry

    jax.lax.fori_loop(0, jmax, body, 0)
    ds_o[...] = acc_ref[:, 0, :]
    drx_o[...] = acc_ref[:, 1, :]
    dry_o[...] = acc_ref[:, 2, :]
    dbw_o[...] = acc_ref[:, 3, :]
    dbh_o[...] = acc_ref[:, 4, :]
    dix_o[...] = acc_ref[:, 5, :]


def _nms_kernel(sc_ref, inds_ref, rx_ref, ry_ref, bw_ref, bh_ref, cx_ref,
                cy_ref, scale_ref, x1o, y1o, x2o, y2o, sco,
                x1s, y1s, x2s, y2s, ars, kps, W, H, K):
    scores = sc_ref[...]
    inds = inds_ref[...]
    ys = (inds // W).astype(jnp.float32)
    xs = (inds % W).astype(jnp.float32)
    xs = xs + rx_ref[...]
    ys = ys + ry_ref[...]
    bw = bw_ref[...]
    bh = bh_ref[...]
    x1 = xs - bw * 0.5
    y1 = ys - bh * 0.5
    x2 = xs + bw * 0.5
    y2 = ys + bh * 0.5
    cx = cx_ref[...]
    cy = cy_ref[...]
    scale = scale_ref[...]
    x1 = (x1 - W / 2.0) * scale + cx
    x2 = (x2 - W / 2.0) * scale + cx
    y1 = (y1 - H / 2.0) * scale + cy
    y2 = (y2 - H / 2.0) * scale + cy
    x1s[...] = x1
    y1s[...] = y1
    x2s[...] = x2
    y2s[...] = y2
    ars[...] = jnp.clip(x2 - x1, 0.0) * jnp.clip(y2 - y1, 0.0)
    kps[...] = (scores >= OUT_THRESH).astype(jnp.float32)
    B = scores.shape[0]
    lane128 = jax.lax.broadcasted_iota(jnp.int32, (B, 128), 1)

    for c in range(KPAD // 128):
        CS = c * 128
        TS = KPAD - CS
        lane_tail = jax.lax.broadcasted_iota(jnp.int32, (B, TS), 1) + CS

        def body(il, carry, CS=CS, TS=TS, lane_tail=lane_tail):
            sel = lane128 == il

            def ext(ref):
                cc = ref[:, CS:CS + 128]
                return jnp.max(jnp.where(sel, cc, -3.4e38), axis=1,
                               keepdims=True)

            kx1 = ext(x1s)
            ky1 = ext(y1s)
            kx2 = ext(x2s)
            ky2 = ext(y2s)
            kar = ext(ars)
            kpi = ext(kps)
            ix1 = jnp.maximum(x1s[:, CS:], kx1)
            iy1 = jnp.maximum(y1s[:, CS:], ky1)
            ix2 = jnp.minimum(x2s[:, CS:], kx2)
            iy2 = jnp.minimum(y2s[:, CS:], ky2)
            inter = jnp.clip(ix2 - ix1, 0.0) * jnp.clip(iy2 - iy1, 0.0)
            union = jnp.maximum(ars[:, CS:] + kar - inter, 1e-6)
            sup = (kpi > 0.0) & (inter > IOU_THR * union) & (lane_tail >
                                                             CS + il)
            kps[:, CS:] = kps[:, CS:] * (1.0 - sup.astype(jnp.float32))
            return carry

        jax.lax.fori_loop(0, 128, body, 0)
    kp = kps[...]
    x1o[...] = x1 * kp
    y1o[...] = y1 * kp
    x2o[...] = x2 * kp
    y2o[...] = y2 * kp
    sco[...] = scores * kp


def kernel(hm, wh, reg, target_sizes):
    B, C, H, W = hm.shape
    K = 1000
    HW = H * W
    RPAD = 328

    heat, thr = pl.pallas_call(
        functools.partial(_heat_kernel, K=K),
        out_shape=[jax.ShapeDtypeStruct((B, H, W), jnp.float32),
                   jax.ShapeDtypeStruct((B, 1), jnp.float32)],
    )(hm.reshape(B, H, W))

    def to_grid(a):
        flat = a.reshape(B, HW)
        flat = jnp.pad(flat, ((0, 0), (0, RPAD * 128 - HW)))
        return flat.reshape(B, RPAD, 128)

    regf = reg.reshape(B, 2, HW)
    whf = wh.reshape(B, 2, HW)
    ds, drx, dry, dbw, dbh, dix = pl.pallas_call(
        _compact_kernel,
        out_shape=[jax.ShapeDtypeStruct((B, KPAD), jnp.float32)] * 6,
        scratch_shapes=[pltpu.VMEM((B, 8, KPAD), jnp.float32)],
    )(to_grid(heat), to_grid(regf[:, 0]), to_grid(regf[:, 1]),
      to_grid(whf[:, 0]), to_grid(whf[:, 1]), thr)

    scores, pos = jax.lax.top_k(ds, K)
    pad = KPAD - K

    def takep(a):
        return jnp.pad(jnp.take_along_axis(a, pos, axis=1),
                       ((0, 0), (0, pad)))

    scores = jnp.pad(scores, ((0, 0), (0, pad)))
    inds = takep(dix).astype(jnp.int32)
    rx, ry, bw, bh = takep(drx), takep(dry), takep(dbw), takep(dbh)

    ts = target_sizes.astype(jnp.float32)
    cx = ts[:, 1:2] / 2.0
    cy = ts[:, 0:1] / 2.0
    scale = jnp.maximum(ts[:, 0:1], ts[:, 1:2]) / float(W)

    out_sh = jax.ShapeDtypeStruct((B, KPAD), jnp.float32)
    scr = pltpu.VMEM((B, KPAD), jnp.float32)
    x1, y1, x2, y2, sc = pl.pallas_call(
        functools.partial(_nms_kernel, W=W, H=H, K=K),
        out_shape=[out_sh] * 5,
        scratch_shapes=[scr] * 6,
    )(scores, inds, rx, ry, bw, bh, cx, cy, scale)

    out = jnp.stack([x1[:, :K], y1[:, :K], x2[:, :K], y2[:, :K], sc[:, :K]],
                    axis=-1)
    return out
